# Initial kernel scaffold; baseline (speedup 1.0000x reference)
import functools

import jax
import jax.numpy as jnp
from jax import lax
from jax.experimental import pallas as pl
from jax.experimental.pallas import tpu as pltpu
from jax.experimental.pallas import tpu_sc as plsc

_N = 10000
_E = 160000
_D = 256
_R = 4
_NC, _NS, _L = 2, 16, 16
_NW = _NC * _NS
_C = 128
_NCHUNK = _E // _C
_TMAX = -(-_NCHUNK // _NW)


def _prescale(h, w_relation):
    bn = 1000

    def body(h_ref, w_ref, out_ref):
        out_ref[...] = (h_ref[...] * w_ref[...])[None]

    return pl.pallas_call(
        body,
        grid=(_R, _N // bn),
        in_specs=[
            pl.BlockSpec((bn, _D), lambda r, i: (i, 0)),
            pl.BlockSpec((1, _D), lambda r, i: (r, 0)),
        ],
        out_specs=pl.BlockSpec((1, bn, _D), lambda r, i: (r, i, 0)),
        out_shape=jax.ShapeDtypeStruct((_R, _N, _D), jnp.float32),
    )(h, w_relation)


_mesh = plsc.VectorSubcoreMesh(
    core_axis_name="c", subcore_axis_name="s", num_cores=_NC, num_subcores=_NS
)


@functools.partial(
    pl.kernel,
    out_type=jax.ShapeDtypeStruct((_E,), jnp.float32),
    mesh=_mesh,
    scratch_types=[
        pltpu.VMEM((_C,), jnp.int32),
        pltpu.VMEM((_C,), jnp.int32),
        pltpu.VMEM((_C,), jnp.int32),
        pltpu.VMEM((_C, _D), jnp.float32),
        pltpu.VMEM((_C, _D), jnp.float32),
        pltpu.VMEM((_C,), jnp.float32),
        pltpu.SemaphoreType.DMA,
        pltpu.SemaphoreType.DMA,
    ],
)
def _distmult_sc(hw_hbm, h_hbm, src_hbm, dst_hbm, et_hbm, out_hbm,
                 sidx, didx, etv, s_rows, d_rows, outv, sem_s, sem_d):
    wid = lax.axis_index("s") * _NC + lax.axis_index("c")
    lane = lax.iota(jnp.int32, _L)

    @pl.loop(0, _TMAX)
    def _round(t):
        c = wid + t * _NW

        @pl.when(c < _NCHUNK)
        def _chunk():
            eoff = pl.multiple_of(c * _C, _C)
            pltpu.sync_copy(src_hbm.at[pl.ds(eoff, _C)], sidx)
            pltpu.sync_copy(dst_hbm.at[pl.ds(eoff, _C)], didx)
            pltpu.sync_copy(et_hbm.at[pl.ds(eoff, _C)], etv)
            for q in range(_C // _L):
                sl = pl.ds(q * _L, _L)
                sidx[sl] = etv[sl] * _N + sidx[sl]
            cp_s = pltpu.async_copy(hw_hbm.at[sidx], s_rows, sem_s)
            cp_d = pltpu.async_copy(h_hbm.at[didx], d_rows, sem_d)
            cp_s.wait()
            cp_d.wait()

            @pl.loop(0, _C // _L)
            def _group(g):
                res = jnp.zeros((_L,), jnp.float32)
                for e2 in range(_L):
                    row = g * _L + e2
                    acc = s_rows[row, pl.ds(0, _L)] * d_rows[row, pl.ds(0, _L)]
                    for k in range(1, _D // _L):
                        sl = pl.ds(k * _L, _L)
                        acc = acc + s_rows[row, sl] * d_rows[row, sl]
                    tot = jnp.sum(acc)
                    res = jnp.where(lane == e2, tot, res)
                outv[pl.ds(pl.multiple_of(g * _L, _L), _L)] = res

            pltpu.sync_copy(outv, out_hbm.at[pl.ds(eoff, _C)])


def kernel(h, edge_index, edge_type, w_relation):
    src = edge_index[0].astype(jnp.int32)
    dst = edge_index[1].astype(jnp.int32)
    et = edge_type.astype(jnp.int32)
    hw = _prescale(h, w_relation).reshape(_R * _N, _D)
    return _distmult_sc(hw, h, src, dst, et)


# speedup vs baseline: 2.3856x; 2.3856x over previous
import functools

import jax
import jax.numpy as jnp
from jax import lax
from jax.experimental import pallas as pl
from jax.experimental.pallas import tpu as pltpu
from jax.experimental.pallas import tpu_sc as plsc

_N = 10000
_E = 160000
_D = 256
_R = 4
_NC, _NS, _L = 2, 16, 16
_NW = _NC * _NS
_C = 128
_NCHUNK = _E // _C
_TMAX = -(-_NCHUNK // _NW)


def _prescale(h, w_relation):
    bn = 1000

    def body(h_ref, w_ref, out_ref):
        r = pl.program_id(0)
        out_ref[...] = (h_ref[...] * w_ref[pl.ds(r, 1), :])[None]

    return pl.pallas_call(
        body,
        grid=(_R, _N // bn),
        in_specs=[
            pl.BlockSpec((bn, _D), lambda r, i: (i, 0)),
            pl.BlockSpec((_R, _D), lambda r, i: (0, 0)),
        ],
        out_specs=pl.BlockSpec((1, bn, _D), lambda r, i: (r, i, 0)),
        out_shape=jax.ShapeDtypeStruct((_R, _N, _D), jnp.float32),
    )(h, w_relation)


_mesh = plsc.VectorSubcoreMesh(
    core_axis_name="c", subcore_axis_name="s", num_cores=_NC, num_subcores=_NS
)


@functools.partial(
    pl.kernel,
    out_type=jax.ShapeDtypeStruct((_E,), jnp.float32),
    mesh=_mesh,
    scratch_types=[
        pltpu.VMEM((_C,), jnp.int32),
        pltpu.VMEM((_C,), jnp.int32),
        pltpu.VMEM((_C,), jnp.int32),
        pltpu.VMEM((_C, _D), jnp.float32),
        pltpu.VMEM((_C, _D), jnp.float32),
        pltpu.VMEM((_C,), jnp.float32),
        pltpu.SemaphoreType.DMA,
        pltpu.SemaphoreType.DMA,
    ],
)
def _distmult_sc(hw_hbm, h_hbm, src_hbm, dst_hbm, et_hbm, out_hbm,
                 sidx, didx, etv, s_rows, d_rows, outv, sem_s, sem_d):
    wid = lax.axis_index("s") * _NC + lax.axis_index("c")
    lane = lax.iota(jnp.int32, _L)

    @pl.loop(0, _TMAX)
    def _round(t):
        c = wid + t * _NW

        @pl.when(c < _NCHUNK)
        def _chunk():
            eoff = pl.multiple_of(c * _C, _C)
            pltpu.sync_copy(src_hbm.at[pl.ds(eoff, _C)], sidx)
            pltpu.sync_copy(dst_hbm.at[pl.ds(eoff, _C)], didx)
            pltpu.sync_copy(et_hbm.at[pl.ds(eoff, _C)], etv)
            for q in range(_C // _L):
                sl = pl.ds(q * _L, _L)
                sidx[sl] = etv[sl] * _N + sidx[sl]
            cp_s = pltpu.async_copy(hw_hbm.at[sidx], s_rows, sem_s)
            cp_d = pltpu.async_copy(h_hbm.at[didx], d_rows, sem_d)
            cp_s.wait()
            cp_d.wait()

            @pl.loop(0, _C // _L)
            def _group(g):
                res = jnp.zeros((_L,), jnp.float32)
                for e2 in range(_L):
                    row = g * _L + e2
                    acc = s_rows[row, pl.ds(0, _L)] * d_rows[row, pl.ds(0, _L)]
                    for k in range(1, _D // _L):
                        sl = pl.ds(k * _L, _L)
                        acc = acc + s_rows[row, sl] * d_rows[row, sl]
                    for sh in (8, 4, 2, 1):
                        perm = jnp.bitwise_xor(lane, sh)
                        acc = acc + jnp.take_along_axis(
                            acc, perm, axis=0, mode="promise_in_bounds")
                    res = jnp.where(lane == e2, acc, res)
                outv[pl.ds(pl.multiple_of(g * _L, _L), _L)] = res

            pltpu.sync_copy(outv, out_hbm.at[pl.ds(eoff, _C)])


def kernel(h, edge_index, edge_type, w_relation):
    src = edge_index[0].astype(jnp.int32)
    dst = edge_index[1].astype(jnp.int32)
    et = edge_type.astype(jnp.int32)
    hw = _prescale(h, w_relation).reshape(_R * _N, _D)
    return _distmult_sc(hw, h, src, dst, et)
